# baseline (speedup 1.0000x reference)
import functools

import jax
import jax.numpy as jnp
from jax.experimental import pallas as pl
from jax.experimental.pallas import tpu as pltpu

_EPS = 1e-10
_BANDWIDTH = (1.0, 1.0)


def _hist_body(xt_ref, sc_ref, o_ref, *, inner):
    i = pl.program_id(1)
    vx = xt_ref[0:1, :]
    vy = xt_ref[1:2, :]
    bx = sc_ref[:, 0:1]
    by = sc_ref[:, 1:2]
    dx = vx - bx
    dy = vy - by
    kx = jnp.exp2(-((dx * dx).astype(jnp.bfloat16))).astype(jnp.float8_e4m3fn)
    ky = jnp.exp2(-((dy * dy).astype(jnp.bfloat16))).astype(jnp.float8_e4m3fn)
    p = jax.lax.dot_general(
        kx, ky, (((1,), (1,)), ((), ())), preferred_element_type=jnp.float32
    )

    @pl.when(i == 0)
    def _init():
        o_ref[...] = jnp.zeros_like(o_ref)

    o_ref[...] += p[None]


def _combine_body(p_ref, o_ref):
    t = p_ref[0] + p_ref[1]
    o_ref[...] = t / (jnp.sum(t) + _EPS)


def kernel(x, bin_edges_x, bin_edges_y):
    n = x.shape[0]
    nb = bin_edges_x.shape[0] - 1
    cx = 0.5 * (bin_edges_x[:-1] + bin_edges_x[1:])
    cy = 0.5 * (bin_edges_y[:-1] + bin_edges_y[1:])
    sx = _BANDWIDTH[0] * (bin_edges_x[1] - bin_edges_x[0])
    sy = _BANDWIDTH[1] * (bin_edges_y[1] - bin_edges_y[0])
    root = jnp.sqrt(jnp.float32(0.5 / jnp.log(2.0)))
    ax = root / sx
    ay = root / sy
    sc = jnp.stack([cx * ax, cy * ay], axis=1)

    chunk = 65536
    nsteps = pl.cdiv(n, chunk)
    inner = pl.cdiv(nsteps, 2)
    total = 2 * inner * chunk
    xt = jnp.pad(
        (x[:, :2] * jnp.stack([ax, ay])).T,
        ((0, 0), (0, total - n)),
        constant_values=1e9,
    )

    body = functools.partial(_hist_body, inner=inner)
    partial_out = pl.pallas_call(
        body,
        grid=(2, inner),
        in_specs=[
            pl.BlockSpec((2, chunk), lambda c, i: (0, c * inner + i)),
            pl.BlockSpec((nb, 2), lambda c, i: (0, 0)),
        ],
        out_specs=pl.BlockSpec((1, nb, nb), lambda c, i: (c, 0, 0)),
        out_shape=jax.ShapeDtypeStruct((2, nb, nb), jnp.float32),
        compiler_params=pltpu.CompilerParams(
            dimension_semantics=("parallel", "arbitrary")
        ),
    )(xt, sc)
    out = pl.pallas_call(
        _combine_body,
        in_specs=[pl.BlockSpec((2, nb, nb), lambda: (0, 0, 0))],
        out_specs=pl.BlockSpec((nb, nb), lambda: (0, 0)),
        out_shape=jax.ShapeDtypeStruct((nb, nb), jnp.float32),
    )(partial_out)
    return out
